# baseline (speedup 1.0000x reference)
import functools
import math

import jax
import jax.numpy as jnp
from jax.experimental import pallas as pl
from jax.experimental.pallas import tpu as pltpu
from jax.experimental.pallas import tpu_sc as plsc

N = 10000
D = 128
K = 27
BN = 512
NPAD = 12288
TOTAL = NPAD * K
GW = 128


def _prep_body(emb27_ref, sh27_ref, weight_ref, wsc_ref, o_ref):
    w = jnp.dot(emb27_ref[...], weight_ref[...],
                preferred_element_type=jnp.float32)
    scale = 1.0 / (K * math.sqrt(float(D)))
    o_ref[...] = w * sh27_ref[...] * scale
    o_ref[13:14, :] = o_ref[13:14, :] + wsc_ref[...] * (1.0 / math.sqrt(float(D)))


def _prep(emb27, sh27, weight, wsc_row):
    return pl.pallas_call(
        _prep_body,
        out_shape=jax.ShapeDtypeStruct((K, D * D), jnp.float32),
    )(emb27, sh27, weight, wsc_row)


def _sc_gather(x_pad, idx_flat):
    mesh = plsc.VectorSubcoreMesh(core_axis_name="c", subcore_axis_name="s")
    out_type = jax.ShapeDtypeStruct((TOTAL, D), x_pad.dtype)

    @functools.partial(pl.kernel, out_type=out_type, mesh=mesh)
    def gather_kernel(x_hbm, i_hbm, o_hbm):
        def body(i_vmem, o_vmem):
            pltpu.sync_copy(x_hbm.at[i_vmem.at[0]], o_vmem)

        pltpu.emit_pipeline(
            body,
            grid=(TOTAL // GW,),
            in_specs=[pl.BlockSpec((1, GW), index_map=lambda i: (0, i))],
            out_specs=[pl.BlockSpec((GW, D), index_map=lambda i: (i, 0))],
            core_axis_name=("c", "s"),
            dimension_semantics=(pltpu.PARALLEL,),
        )(i_hbm, o_hbm)

    return gather_kernel(x_pad, idx_flat)


def _conv_body(g_ref, a_ref, o_ref):
    o_ref[...] = jnp.dot(g_ref[...], a_ref[...],
                         preferred_element_type=jnp.float32)


def _conv(G, Astack):
    return pl.pallas_call(
        _conv_body,
        grid=(NPAD // BN,),
        in_specs=[
            pl.BlockSpec((BN, K * D), lambda i: (i, 0)),
            pl.BlockSpec((K * D, D), lambda i: (0, 0)),
        ],
        out_specs=pl.BlockSpec((BN, D), lambda i: (i, 0)),
        out_shape=jax.ShapeDtypeStruct((NPAD, D), jnp.float32),
    )(G, Astack)


def kernel(x, W_sc, weight, emb, sh, neigh_idx):
    x = x.astype(jnp.float32)
    x_pad = jnp.concatenate([x, jnp.zeros((1, D), x.dtype)], axis=0)
    idx = neigh_idx.astype(jnp.int32)
    idx_flat = jnp.pad(idx, ((0, NPAD - N), (0, 0)),
                       constant_values=N).reshape(1, TOTAL)
    emb27 = emb.transpose(2, 1, 0, 3).reshape(K, -1)
    sh27 = sh[..., 0].transpose(2, 1, 0).reshape(K, 1)
    wsc_row = W_sc.reshape(1, D * D)

    Astack = _prep(emb27, sh27, weight, wsc_row).reshape(K * D, D)
    G = _sc_gather(x_pad, idx_flat).reshape(NPAD, K * D)
    out = _conv(G, Astack)
    return out[:N]


# speedup vs baseline: 1.2253x; 1.2253x over previous
import functools
import math

import jax
import jax.numpy as jnp
from jax.experimental import pallas as pl
from jax.experimental.pallas import tpu as pltpu
from jax.experimental.pallas import tpu_sc as plsc

N = 10000
D = 128
K = 27
BN = 512
NPAD = 10240
TOTAL = NPAD * K
NW = 32
BPW = TOTAL // NW
CH = 480
NCHUNK = BPW // CH


def _prep_body(emb27_ref, sh27_ref, weight_ref, wsc_ref, o_ref):
    w = jnp.dot(emb27_ref[...], weight_ref[...],
                preferred_element_type=jnp.float32)
    scale = 1.0 / (K * math.sqrt(float(D)))
    o_ref[...] = w * sh27_ref[...] * scale
    o_ref[13:14, :] = o_ref[13:14, :] + wsc_ref[...] * (1.0 / math.sqrt(float(D)))


def _prep(emb27, sh27, weight, wsc_row):
    return pl.pallas_call(
        _prep_body,
        out_shape=jax.ShapeDtypeStruct((K, D * D), jnp.float32),
    )(emb27, sh27, weight, wsc_row)


def _sc_gather(x_pad, idx_flat):
    mesh = plsc.VectorSubcoreMesh(core_axis_name="c", subcore_axis_name="s")
    out_type = jax.ShapeDtypeStruct((TOTAL, D), x_pad.dtype)

    @functools.partial(
        pl.kernel, out_type=out_type, mesh=mesh,
        scratch_types=[
            pltpu.VMEM((CH,), jnp.int32),
            pltpu.VMEM((CH, D), jnp.float32),
            pltpu.SemaphoreType.DMA,
        ],
    )
    def gather_kernel(x_hbm, i_hbm, o_hbm, idx_v, rows_v, sem):
        wid = jax.lax.axis_index("s") * 2 + jax.lax.axis_index("c")
        base_w = wid * BPW

        @pl.loop(0, NCHUNK)
        def _(c):
            base = base_w + c * CH
            pltpu.sync_copy(i_hbm.at[pl.ds(base, CH)], idx_v)
            pltpu.async_copy(x_hbm.at[idx_v], rows_v, sem).wait()
            pltpu.sync_copy(rows_v, o_hbm.at[pl.ds(base, CH)])

    return gather_kernel(x_pad, idx_flat)


def _conv_body(g_ref, a_ref, o_ref):
    o_ref[...] = jnp.dot(g_ref[...], a_ref[...],
                         preferred_element_type=jnp.float32)


def _conv(G, Astack):
    return pl.pallas_call(
        _conv_body,
        grid=(NPAD // BN,),
        in_specs=[
            pl.BlockSpec((BN, K * D), lambda i: (i, 0)),
            pl.BlockSpec((K * D, D), lambda i: (0, 0)),
        ],
        out_specs=pl.BlockSpec((BN, D), lambda i: (i, 0)),
        out_shape=jax.ShapeDtypeStruct((NPAD, D), jnp.float32),
    )(G, Astack)


def kernel(x, W_sc, weight, emb, sh, neigh_idx):
    x = x.astype(jnp.float32)
    x_pad = jnp.concatenate([x, jnp.zeros((1, D), x.dtype)], axis=0)
    idx = neigh_idx.astype(jnp.int32)
    idx_flat = jnp.pad(idx, ((0, NPAD - N), (0, 0)),
                       constant_values=N).reshape(TOTAL)
    emb27 = emb.transpose(2, 1, 0, 3).reshape(K, -1)
    sh27 = sh[..., 0].transpose(2, 1, 0).reshape(K, 1)
    wsc_row = W_sc.reshape(1, D * D)

    Astack = _prep(emb27, sh27, weight, wsc_row).reshape(K * D, D)
    G = _sc_gather(x_pad, idx_flat).reshape(NPAD, K * D)
    out = _conv(G, Astack)
    return out[:N]


# speedup vs baseline: 33.3030x; 27.1794x over previous
import functools
import math

import jax
import jax.numpy as jnp
from jax.experimental import pallas as pl
from jax.experimental.pallas import tpu as pltpu
from jax.experimental.pallas import tpu_sc as plsc

N = 10000
D = 128
K = 27
BN = 512
NPAD = 10240
TOTAL = NPAD * K
NW = 32
BPW = TOTAL // NW
CH = 480
NCHUNK = BPW // CH
NZ = 1024


def _prep_body(emb27_ref, sh27_ref, weight_ref, wsc_ref, o_ref):
    w = jnp.dot(emb27_ref[...], weight_ref[...],
                preferred_element_type=jnp.float32)
    scale = 1.0 / (K * math.sqrt(float(D)))
    o_ref[...] = w * sh27_ref[...] * scale
    o_ref[13:14, :] = o_ref[13:14, :] + wsc_ref[...] * (1.0 / math.sqrt(float(D)))


def _prep(emb27, sh27, weight, wsc_row):
    return pl.pallas_call(
        _prep_body,
        out_shape=jax.ShapeDtypeStruct((K, D * D), jnp.float32),
    )(emb27, sh27, weight, wsc_row)


def _sc_gather(x_pad, idx_flat):
    mesh = plsc.VectorSubcoreMesh(core_axis_name="c", subcore_axis_name="s")
    out_type = jax.ShapeDtypeStruct((TOTAL, D), x_pad.dtype)

    @functools.partial(
        pl.kernel, out_type=out_type, mesh=mesh,
        scratch_types=[
            pltpu.VMEM((CH,), jnp.int32),
            pltpu.VMEM((CH, D), jnp.float32),
            pltpu.SemaphoreType.DMA,
        ],
    )
    def gather_kernel(x_hbm, i_hbm, o_hbm, idx_v, rows_v, sem):
        wid = jax.lax.axis_index("s") * 2 + jax.lax.axis_index("c")
        base_w = wid * BPW

        @pl.loop(0, NCHUNK)
        def _(c):
            base = base_w + c * CH
            pltpu.sync_copy(i_hbm.at[pl.ds(base, CH)], idx_v)
            pltpu.async_copy(x_hbm.at[idx_v], rows_v, sem).wait()
            pltpu.sync_copy(rows_v, o_hbm.at[pl.ds(base, CH)])

    return gather_kernel(x_pad, idx_flat)


def _conv_body(g_ref, a_ref, o_ref):
    o_ref[...] = jnp.dot(g_ref[...], a_ref[...],
                         preferred_element_type=jnp.float32)


def _conv(G, Astack):
    return pl.pallas_call(
        _conv_body,
        grid=(NPAD // BN,),
        in_specs=[
            pl.BlockSpec((BN, K * D), lambda i: (i, 0)),
            pl.BlockSpec((K * D, D), lambda i: (0, 0)),
        ],
        out_specs=pl.BlockSpec((BN, D), lambda i: (i, 0)),
        out_shape=jax.ShapeDtypeStruct((NPAD, D), jnp.float32),
    )(G, Astack)


def kernel(x, W_sc, weight, emb, sh, neigh_idx):
    x = x.astype(jnp.float32)
    x_pad = jnp.concatenate([x, jnp.zeros((NZ, D), x.dtype)], axis=0)
    idx = neigh_idx.astype(jnp.int32)
    idx_flat = jnp.pad(idx, ((0, NPAD - N), (0, 0)),
                       constant_values=N).reshape(TOTAL)
    spread = N + jax.lax.rem(jnp.arange(TOTAL, dtype=jnp.int32),
                             jnp.int32(NZ))
    idx_flat = jnp.where(idx_flat == N, spread, idx_flat)
    emb27 = emb.transpose(2, 1, 0, 3).reshape(K, -1)
    sh27 = sh[..., 0].transpose(2, 1, 0).reshape(K, 1)
    wsc_row = W_sc.reshape(1, D * D)

    Astack = _prep(emb27, sh27, weight, wsc_row).reshape(K * D, D)
    G = _sc_gather(x_pad, idx_flat).reshape(NPAD, K * D)
    out = _conv(G, Astack)
    return out[:N]
